# baseline (speedup 1.0000x reference)
import functools

import jax
import jax.numpy as jnp
from jax import lax
from jax.experimental import pallas as pl
from jax.experimental.pallas import tpu as pltpu
from jax.experimental.pallas import tpu_sc as plsc

NC = 2
NS = 16
NW = NC * NS
LANES = 16
CHUNK = 128
PKBITS = 14


def _pre_body(x_ref, w_ref, al_ref, ar_ref, z_ref, sl_ref, sr_ref, cv_ref):
    x = x_ref[...]
    z = lax.dot_general(x, w_ref[...], (((1,), (1,)), ((), ())),
                        preferred_element_type=jnp.float32)
    z_ref[...] = z
    sl = jnp.sum(z * al_ref[...][None, :], axis=1)
    sr = jnp.sum(z * ar_ref[...][None, :], axis=1)
    sl_ref[...] = sl
    sr_ref[...] = sr
    c_sh = jnp.maximum(jnp.max(sl) + jnp.max(sr), 0.0)
    cv_ref[...] = jnp.full((LANES,), c_sh, jnp.float32)


def _post_body(n, hp_ref, dp_ref, o_ref):
    den = dp_ref[0, :n] + dp_ref[1, :n]
    den = jnp.where(den == 0.0, 1.0, den)
    h = hp_ref[0, :n, :] + hp_ref[1, :n, :]
    o_ref[...] = h / den[:, None]


def _make_sc_kernel(n, d, n_pad, ch):
    rows_per_tile = n_pad // NS
    zcopies = rows_per_tile // CHUNK

    mesh = plsc.VectorSubcoreMesh(core_axis_name="c", subcore_axis_name="s")

    def buf(tp):
        return [tp, tp]

    @functools.partial(
        pl.kernel,
        out_type=[
            jax.ShapeDtypeStruct((NC, n_pad, d), jnp.float32),
            jax.ShapeDtypeStruct((NC, n_pad), jnp.float32),
        ],
        mesh=mesh,
        scratch_types=(
            [pltpu.VMEM((ch + 2, CHUNK), jnp.int32)]
            + buf(pltpu.VMEM((CHUNK,), jnp.int32))
            + buf(pltpu.VMEM((CHUNK,), jnp.int32))
            + buf(pltpu.VMEM((CHUNK,), jnp.float32))
            + buf(pltpu.VMEM((CHUNK,), jnp.float32))
            + buf(pltpu.VMEM((CHUNK,), jnp.float32))
            + buf(pltpu.VMEM((CHUNK,), jnp.float32))
            + buf(pltpu.VMEM((CHUNK, d), jnp.float32))
            + [
                pltpu.VMEM((LANES,), jnp.float32),
                pltpu.VMEM_SHARED((n_pad, d), jnp.float32),
                pltpu.VMEM_SHARED((n_pad,), jnp.float32),
            ]
            + buf(pltpu.SemaphoreType.DMA)
            + buf(pltpu.SemaphoreType.DMA)
            + buf(pltpu.SemaphoreType.DMA)
        ),
    )
    def sc_kernel(z_hbm, sl_hbm, sr_hbm, pk_hbm, w_hbm, cv_hbm,
                  h_out, den_out, *scr):
        pk_v = scr[0]
        src_c = scr[1:3]
        dst_c = scr[3:5]
        w_c = scr[5:7]
        slg = scr[7:9]
        srg = scr[9:11]
        ex_c = scr[11:13]
        rows = scr[13:15]
        cv_v = scr[15]
        h_sh = scr[16]
        den_sh = scr[17]
        sem_w = scr[18:20]
        sem_s = scr[20:22]
        sem_z = scr[22:24]

        c = lax.axis_index("c")
        s = lax.axis_index("s")
        w_id = c * NS + s
        base = s * rows_per_tile

        pltpu.sync_copy(cv_hbm, cv_v)
        c_sh = cv_v[...]
        pltpu.sync_copy(pk_hbm.at[w_id], pk_v)

        def zrow(r, _):
            for f in range(d // LANES):
                rows[0][r, pl.ds(f * LANES, LANES)] = jnp.zeros(
                    (LANES,), jnp.float32)
            return 0
        lax.fori_loop(0, CHUNK, zrow, 0)
        for q in range(zcopies):
            pltpu.sync_copy(rows[0], h_sh.at[pl.ds(base + q * CHUNK, CHUNK)])
            pltpu.sync_copy(rows[0].at[0],
                            den_sh.at[pl.ds(base + q * CHUNK, CHUNK)])

        def prep(b, j):
            for k in range(CHUNK // LANES):
                v = pk_v[j, pl.ds(k * LANES, LANES)]
                src_c[b][pl.ds(k * LANES, LANES)] = lax.shift_right_logical(
                    v, PKBITS)
                dst_c[b][pl.ds(k * LANES, LANES)] = lax.bitwise_and(
                    v, (1 << PKBITS) - 1)
            pltpu.async_copy(w_hbm.at[w_id, j], w_c[b], sem_w[b])
            pltpu.async_copy(sl_hbm.at[src_c[b]], slg[b], sem_s[b])
            pltpu.async_copy(sr_hbm.at[dst_c[b]], srg[b], sem_s[b])
            pltpu.async_copy(z_hbm.at[src_c[b]], rows[b], sem_z[b])

        def wait_in(b):
            pltpu.make_async_copy(
                w_hbm.at[0, 0], w_c[b], sem_w[b]).wait()
            pltpu.make_async_copy(
                sl_hbm.at[pl.ds(0, CHUNK)], slg[b], sem_s[b]).wait()
            pltpu.make_async_copy(
                sl_hbm.at[pl.ds(0, CHUNK)], srg[b], sem_s[b]).wait()
            pltpu.make_async_copy(
                z_hbm.at[pl.ds(0, CHUNK)], rows[b], sem_z[b]).wait()

        def process(b):
            for k in range(CHUNK // LANES):
                wk = w_c[b][pl.ds(k * LANES, LANES)]
                raw = (slg[b][pl.ds(k * LANES, LANES)]
                       + srg[b][pl.ds(k * LANES, LANES)])
                e = wk * jnp.maximum(raw, 0.01 * raw)
                ex = jnp.where(wk >= 0.0, jnp.exp(e - c_sh), 0.0)
                ex_c[b][pl.ds(k * LANES, LANES)] = ex
            pltpu.sync_copy(ex_c[b], den_sh.at[dst_c[b]], add=True)
            def rblk(k, _):
                exk = ex_c[b][pl.ds(k * LANES, LANES)]
                for r in range(LANES):
                    a = exk[r]
                    row = k * LANES + r
                    for f in range(d // LANES):
                        v = rows[b][row, pl.ds(f * LANES, LANES)]
                        rows[b][row, pl.ds(f * LANES, LANES)] = v * a
                return 0
            lax.fori_loop(0, CHUNK // LANES, rblk, 0)
            pltpu.sync_copy(rows[b], h_sh.at[dst_c[b]], add=True)

        prep(0, 0)
        prep(1, 1)
        plsc.subcore_barrier()

        def pair(t, _):
            for b in range(2):
                j = t * 2 + b
                wait_in(b)
                process(b)
                prep(b, j + 2)
            return 0
        lax.fori_loop(0, ch // 2, pair, 0)
        wait_in(0)
        wait_in(1)

        plsc.subcore_barrier()

        pltpu.sync_copy(h_sh.at[pl.ds(base, rows_per_tile)],
                        h_out.at[c, pl.ds(base, rows_per_tile)])
        pltpu.sync_copy(den_sh.at[pl.ds(base, rows_per_tile)],
                        den_out.at[c, pl.ds(base, rows_per_tile)])

    return sc_kernel


def kernel(x, edge_index, edge_weight, W_fc, W_attn):
    n, d_in = x.shape
    d = W_fc.shape[0]
    e_cnt = edge_index.shape[1]
    assert n % LANES == 0 and d % LANES == 0

    a_l = W_attn[0, :d]
    a_r = W_attn[0, d:]

    z, sl, sr, cvec = pl.pallas_call(
        _pre_body,
        out_shape=[
            jax.ShapeDtypeStruct((n, d), jnp.float32),
            jax.ShapeDtypeStruct((n,), jnp.float32),
            jax.ShapeDtypeStruct((n,), jnp.float32),
            jax.ShapeDtypeStruct((LANES,), jnp.float32),
        ],
    )(x, W_fc, a_l, a_r)

    ch = -(-e_cnt // (NW * CHUNK))
    ch += ch % 2
    e_pad = NW * ch * CHUNK
    pk_full = (edge_index[0] << PKBITS) | edge_index[1]
    pk = jnp.concatenate(
        [jnp.pad(pk_full, (0, e_pad - e_cnt)).reshape(NW, ch, CHUNK),
         jnp.zeros((NW, 2, CHUNK), jnp.int32)], axis=1)
    wgt = jnp.concatenate(
        [jnp.pad(edge_weight, (0, e_pad - e_cnt),
                 constant_values=-1.0).reshape(NW, ch, CHUNK),
         jnp.full((NW, 2, CHUNK), -1.0, jnp.float32)], axis=1)

    n_pad = -(-n // (NS * CHUNK)) * NS * CHUNK
    assert n_pad < (1 << PKBITS)
    hp, dp = _make_sc_kernel(n, d, n_pad, ch)(z, sl, sr, pk, wgt, cvec)

    out = pl.pallas_call(
        functools.partial(_post_body, n),
        out_shape=jax.ShapeDtypeStruct((n, d), jnp.float32),
    )(hp, dp)
    return out


# speedup vs baseline: 1.8934x; 1.8934x over previous
import functools

import jax
import jax.numpy as jnp
from jax import lax
from jax.experimental import pallas as pl
from jax.experimental.pallas import tpu as pltpu
from jax.experimental.pallas import tpu_sc as plsc

NC = 2
NS = 16
NW = NC * NS
LANES = 16
CHUNK = 128
PKBITS = 14


def _pre_body(x_ref, w_ref, al_ref, ar_ref, z_ref, sl_ref, sr_ref, cv_ref):
    x = x_ref[...]
    z = lax.dot_general(x, w_ref[...], (((1,), (1,)), ((), ())),
                        preferred_element_type=jnp.float32)
    z_ref[...] = z
    sl = jnp.sum(z * al_ref[...][None, :], axis=1)
    sr = jnp.sum(z * ar_ref[...][None, :], axis=1)
    sl_ref[...] = sl
    sr_ref[...] = sr
    c_sh = jnp.maximum(jnp.max(sl) + jnp.max(sr), 0.0)
    cv_ref[...] = jnp.full((LANES,), c_sh, jnp.float32)


def _post_body(n, hp_ref, dp_ref, o_ref):
    den = dp_ref[0, :n] + dp_ref[1, :n]
    den = jnp.where(den == 0.0, 1.0, den)
    h = hp_ref[0, :n, :] + hp_ref[1, :n, :]
    o_ref[...] = h / den[:, None]


def _make_sc_kernel(n, d, n_pad, ch):
    rows_per_tile = n_pad // NS
    zcopies = rows_per_tile // CHUNK

    mesh = plsc.VectorSubcoreMesh(core_axis_name="c", subcore_axis_name="s")

    @functools.partial(
        pl.kernel,
        out_type=[
            jax.ShapeDtypeStruct((NC, n_pad, d), jnp.float32),
            jax.ShapeDtypeStruct((NC, n_pad), jnp.float32),
        ],
        mesh=mesh,
        scratch_types=[
            pltpu.VMEM((ch, CHUNK), jnp.int32),
            pltpu.VMEM((ch, CHUNK), jnp.float32),
            pltpu.VMEM((CHUNK,), jnp.int32),
            pltpu.VMEM((CHUNK,), jnp.int32),
            pltpu.VMEM((CHUNK,), jnp.float32),
            pltpu.VMEM((CHUNK,), jnp.float32),
            pltpu.VMEM((CHUNK,), jnp.float32),
            pltpu.VMEM((CHUNK, d), jnp.float32),
            pltpu.VMEM((LANES,), jnp.float32),
            pltpu.VMEM_SHARED((n_pad, d), jnp.float32),
            pltpu.VMEM_SHARED((n_pad,), jnp.float32),
            pltpu.SemaphoreType.DMA,
            pltpu.SemaphoreType.DMA,
        ],
    )
    def sc_kernel(z_hbm, sl_hbm, sr_hbm, pk_hbm, w_hbm, cv_hbm,
                  h_out, den_out,
                  pk_v, w_v, src_c, dst_c, slg, srg, ex_c, rows, cv_v,
                  h_sh, den_sh, sem_s, sem_z):
        c = lax.axis_index("c")
        s = lax.axis_index("s")
        w_id = c * NS + s
        base = s * rows_per_tile

        pltpu.sync_copy(cv_hbm, cv_v)
        c_sh = cv_v[...]
        pltpu.sync_copy(pk_hbm.at[w_id], pk_v)
        pltpu.sync_copy(w_hbm.at[w_id], w_v)

        def zrow(r, _):
            for f in range(d // LANES):
                rows[r, pl.ds(f * LANES, LANES)] = jnp.zeros(
                    (LANES,), jnp.float32)
            return 0
        lax.fori_loop(0, CHUNK, zrow, 0)
        for q in range(zcopies):
            pltpu.sync_copy(rows, h_sh.at[pl.ds(base + q * CHUNK, CHUNK)])
            pltpu.sync_copy(rows.at[0],
                            den_sh.at[pl.ds(base + q * CHUNK, CHUNK)])

        plsc.subcore_barrier()

        def cbody(j, _):
            for k in range(CHUNK // LANES):
                v = pk_v[j, pl.ds(k * LANES, LANES)]
                src_c[pl.ds(k * LANES, LANES)] = lax.shift_right_logical(
                    v, PKBITS)
                dst_c[pl.ds(k * LANES, LANES)] = lax.bitwise_and(
                    v, (1 << PKBITS) - 1)
            zcp = pltpu.async_copy(z_hbm.at[src_c], rows, sem_z)
            ga = pltpu.async_copy(sl_hbm.at[src_c], slg, sem_s)
            gb = pltpu.async_copy(sr_hbm.at[dst_c], srg, sem_s)
            ga.wait()
            gb.wait()
            for k in range(CHUNK // LANES):
                wk = w_v[j, pl.ds(k * LANES, LANES)]
                raw = (slg[pl.ds(k * LANES, LANES)]
                       + srg[pl.ds(k * LANES, LANES)])
                e = wk * jnp.maximum(raw, 0.01 * raw)
                ex = jnp.where(wk >= 0.0, jnp.exp(e - c_sh), 0.0)
                ex_c[pl.ds(k * LANES, LANES)] = ex
            pltpu.sync_copy(ex_c, den_sh.at[dst_c], add=True)
            zcp.wait()
            def rblk(k, _):
                exk = ex_c[pl.ds(k * LANES, LANES)]
                for r in range(LANES):
                    a = exk[r]
                    row = k * LANES + r
                    for f in range(d // LANES):
                        v = rows[row, pl.ds(f * LANES, LANES)]
                        rows[row, pl.ds(f * LANES, LANES)] = v * a
                return 0
            lax.fori_loop(0, CHUNK // LANES, rblk, 0)
            pltpu.sync_copy(rows, h_sh.at[dst_c], add=True)
            return 0
        lax.fori_loop(0, ch, cbody, 0)

        plsc.subcore_barrier()

        pltpu.sync_copy(h_sh.at[pl.ds(base, rows_per_tile)],
                        h_out.at[c, pl.ds(base, rows_per_tile)])
        pltpu.sync_copy(den_sh.at[pl.ds(base, rows_per_tile)],
                        den_out.at[c, pl.ds(base, rows_per_tile)])

    return sc_kernel


def kernel(x, edge_index, edge_weight, W_fc, W_attn):
    n, d_in = x.shape
    d = W_fc.shape[0]
    e_cnt = edge_index.shape[1]
    assert n % LANES == 0 and d % LANES == 0

    a_l = W_attn[0, :d]
    a_r = W_attn[0, d:]

    z, sl, sr, cvec = pl.pallas_call(
        _pre_body,
        out_shape=[
            jax.ShapeDtypeStruct((n, d), jnp.float32),
            jax.ShapeDtypeStruct((n,), jnp.float32),
            jax.ShapeDtypeStruct((n,), jnp.float32),
            jax.ShapeDtypeStruct((LANES,), jnp.float32),
        ],
    )(x, W_fc, a_l, a_r)

    ch = -(-e_cnt // (NW * CHUNK))
    e_pad = NW * ch * CHUNK
    pk_full = (edge_index[0] << PKBITS) | edge_index[1]
    pk = jnp.pad(pk_full, (0, e_pad - e_cnt)).reshape(NW, ch, CHUNK)
    wgt = jnp.pad(edge_weight, (0, e_pad - e_cnt),
                  constant_values=-1.0).reshape(NW, ch, CHUNK)

    n_pad = -(-n // (NS * CHUNK)) * NS * CHUNK
    assert n_pad < (1 << PKBITS)
    hp, dp = _make_sc_kernel(n, d, n_pad, ch)(z, sl, sr, pk, wgt, cvec)

    out = pl.pallas_call(
        functools.partial(_post_body, n),
        out_shape=jax.ShapeDtypeStruct((n, d), jnp.float32),
    )(hp, dp)
    return out
